# baseline (device time: 21784 ns/iter reference)
import jax
import jax.numpy as jnp
from jax import lax
from jax.experimental import pallas as pl
from jax.experimental.pallas import tpu as pltpu

N_DEV = 4
B_LOC = 2
SQ = 256
SKV = 256
HQ = 4
DH = 64
D_MODEL = 512
D_CHUNK = HQ * DH
HDC = D_CHUNK // 2


def _body(x_ref, wq_ref, wo_ref, k_ref, v_ref, out_ref,
          wqb, wob, commq, commo, kperm, vperm,
          sendq, sendo, recvq, recvo):
    my = lax.axis_index("i")
    right = (my + 1) % N_DEV
    left = (my + N_DEV - 1) % N_DEV
    diag = (my + 2) % N_DEV

    barrier = pltpu.get_barrier_semaphore()
    for nbr in (left, right, diag):
        pl.semaphore_signal(barrier, inc=1, device_id=(nbr,),
                            device_id_type=pl.DeviceIdType.MESH)
    pl.semaphore_wait(barrier, 3)

    for h in range(2):
        wqb[h] = wq_ref[:, h * HDC:(h + 1) * HDC].astype(jnp.bfloat16)
        wob[h] = wo_ref[h * HDC:(h + 1) * HDC, :].astype(jnp.bfloat16)
    sends = []

    def send_chunk(slot, tgt):
        for buf, comm, ssem, rsem in ((wqb, commq, sendq, recvq),
                                      (wob, commo, sendo, recvo)):
            for h in range(2):
                rdma = pltpu.make_async_remote_copy(
                    src_ref=buf.at[h],
                    dst_ref=comm.at[slot, h],
                    send_sem=ssem.at[slot, h],
                    recv_sem=rsem.at[slot, h],
                    device_id=(tgt,),
                    device_id_type=pl.DeviceIdType.MESH,
                )
                rdma.start()
                sends.append(rdma)

    send_chunk(0, right)
    send_chunk(1, left)

    for m in range(N_DEV):
        orig = [m, (m - 1) % N_DEV, (m + 1) % N_DEV, (m + 2) % N_DEV]

        @pl.when(my == m)
        def _(orig=orig):
            for b in range(B_LOC):
                for p in range(N_DEV):
                    o = orig[p]
                    kperm[b, :, p * D_CHUNK:(p + 1) * D_CHUNK] = (
                        k_ref[b, :, o * D_CHUNK:(o + 1) * D_CHUNK])
                    vperm[b, :, p * D_CHUNK:(p + 1) * D_CHUNK] = (
                        v_ref[b, :, o * D_CHUNK:(o + 1) * D_CHUNK])

    xb = x_ref[...].reshape(B_LOC * SQ, D_MODEL).astype(jnp.bfloat16)

    qi = lax.broadcasted_iota(jnp.int32, (SQ, SKV), 0)
    ki = lax.broadcasted_iota(jnp.int32, (SQ, SKV), 1)
    mask = (jnp.abs(qi - ki) <= 128) | (ki < 32) | (qi < 32)
    bias = jnp.where(mask, 0.0, -1e9).astype(jnp.float32)

    def attn_half(p, h, wq_h):
        q = lax.dot_general(xb, wq_h, (((1,), (0,)), ((), ())),
                            preferred_element_type=jnp.float32)
        qb = (q * 0.125).astype(jnp.bfloat16)
        ctxbs = []
        for b in range(B_LOC):
            ctx_parts = []
            for j in range(2):
                hh = 2 * h + j
                qs = qb[b * SQ:(b + 1) * SQ, j * DH:(j + 1) * DH]
                base = p * D_CHUNK + hh * DH
                k = kperm[b, :, base:base + DH]
                s = lax.dot_general(qs, k, (((1,), (1,)), ((), ())),
                                    preferred_element_type=jnp.float32)
                w = jnp.exp(s + bias)
                den = jnp.sum(w, axis=1, keepdims=True)
                wb = w.astype(jnp.bfloat16)
                v = vperm[b, :, base:base + DH]
                ctx = lax.dot_general(wb, v, (((1,), (0,)), ((), ())),
                                      preferred_element_type=jnp.float32)
                ctx_parts.append((ctx / den).astype(jnp.bfloat16))
            ctxbs.append(jnp.concatenate(ctx_parts, axis=1))
        return ctxbs

    def out_half(p, h, wo_h, ctxbs):
        for b in range(B_LOC):
            o = lax.dot_general(ctxbs[b], wo_h, (((1,), (0,)), ((), ())),
                                preferred_element_type=jnp.float32)
            if p == 0 and h == 0:
                out_ref[b] = o
            else:
                out_ref[b] += o

    for h in range(2):
        out_half(0, h, wob[h], attn_half(0, h, wqb[h]))
    send_chunk(2, diag)

    def recv_wait(comm, ssem, rsem, slot, h):
        recv = pltpu.make_async_remote_copy(
            src_ref=wqb.at[0] if comm is commq else wob.at[0],
            dst_ref=comm.at[slot, h],
            send_sem=ssem.at[slot, h],
            recv_sem=rsem.at[slot, h],
            device_id=(left,),
            device_id_type=pl.DeviceIdType.MESH,
        )
        recv.wait_recv()

    for p in range(1, N_DEV):
        slot = p - 1
        for h in range(2):
            recv_wait(commq, sendq, recvq, slot, h)
            ctxbs = attn_half(p, h, commq[slot, h])
            recv_wait(commo, sendo, recvo, slot, h)
            out_half(p, h, commo[slot, h], ctxbs)

    for rdma in sends:
        rdma.wait_send()


def kernel(x, Wq, K_ext, V_ext, Wo):
    my = lax.axis_index("i")

    def prep(t):
        t = lax.dynamic_slice_in_dim(t, my * B_LOC, B_LOC, 0)
        return t.astype(jnp.bfloat16).reshape(B_LOC, SKV, N_DEV * D_CHUNK)

    k2 = prep(K_ext)
    v2 = prep(V_ext)

    return pl.pallas_call(
        _body,
        out_shape=jax.ShapeDtypeStruct((B_LOC, SQ, D_MODEL), jnp.float32),
        in_specs=[
            pl.BlockSpec(memory_space=pltpu.VMEM),
            pl.BlockSpec(memory_space=pltpu.VMEM),
            pl.BlockSpec(memory_space=pltpu.VMEM),
            pl.BlockSpec(memory_space=pltpu.VMEM),
            pl.BlockSpec(memory_space=pltpu.VMEM),
        ],
        out_specs=pl.BlockSpec(memory_space=pltpu.VMEM),
        scratch_shapes=[
            pltpu.VMEM((2, D_MODEL, HDC), jnp.bfloat16),
            pltpu.VMEM((2, HDC, D_MODEL), jnp.bfloat16),
            pltpu.VMEM((3, 2, D_MODEL, HDC), jnp.bfloat16),
            pltpu.VMEM((3, 2, HDC, D_MODEL), jnp.bfloat16),
            pltpu.VMEM((B_LOC, SKV, N_DEV * D_CHUNK), jnp.bfloat16),
            pltpu.VMEM((B_LOC, SKV, N_DEV * D_CHUNK), jnp.bfloat16),
            pltpu.SemaphoreType.DMA((3, 2)),
            pltpu.SemaphoreType.DMA((3, 2)),
            pltpu.SemaphoreType.DMA((3, 2)),
            pltpu.SemaphoreType.DMA((3, 2)),
        ],
        compiler_params=pltpu.CompilerParams(collective_id=0),
    )(x, Wq, Wo, k2, v2)


# device time: 21405 ns/iter; 1.0177x vs baseline; 1.0177x over previous
import jax
import jax.numpy as jnp
from jax import lax
from jax.experimental import pallas as pl
from jax.experimental.pallas import tpu as pltpu

N_DEV = 4
B_LOC = 2
SQ = 256
SKV = 256
HQ = 4
DH = 64
D_MODEL = 512
D_CHUNK = HQ * DH


def _body(x_ref, wq_ref, wo_ref, k_ref, v_ref, out_ref,
          wqb, wob, commq, commo, kperm, vperm,
          sendq, sendo, recvq, recvo):
    my = lax.axis_index("i")
    right = (my + 1) % N_DEV
    left = (my + N_DEV - 1) % N_DEV
    diag = (my + 2) % N_DEV

    barrier = pltpu.get_barrier_semaphore()
    for nbr in (left, right, diag):
        pl.semaphore_signal(barrier, inc=1, device_id=(nbr,),
                            device_id_type=pl.DeviceIdType.MESH)
    pl.semaphore_wait(barrier, 3)

    wqb[...] = wq_ref[...].astype(jnp.bfloat16)
    wob[...] = wo_ref[...].astype(jnp.bfloat16)
    sends = []

    def send_chunk(slot, tgt):
        for buf, comm, ssem, rsem in ((wqb, commq, sendq, recvq),
                                      (wob, commo, sendo, recvo)):
            rdma = pltpu.make_async_remote_copy(
                src_ref=buf,
                dst_ref=comm.at[slot],
                send_sem=ssem.at[slot],
                recv_sem=rsem.at[slot],
                device_id=(tgt,),
                device_id_type=pl.DeviceIdType.MESH,
            )
            rdma.start()
            sends.append(rdma)

    send_chunk(0, right)
    send_chunk(1, left)

    for m in range(N_DEV):
        orig = [m, (m - 1) % N_DEV, (m + 1) % N_DEV, (m + 2) % N_DEV]

        @pl.when(my == m)
        def _(orig=orig):
            for b in range(B_LOC):
                for p in range(N_DEV):
                    o = orig[p]
                    kperm[b, :, p * D_CHUNK:(p + 1) * D_CHUNK] = (
                        k_ref[b, :, o * D_CHUNK:(o + 1) * D_CHUNK])
                    vperm[b, :, p * D_CHUNK:(p + 1) * D_CHUNK] = (
                        v_ref[b, :, o * D_CHUNK:(o + 1) * D_CHUNK])

    xb = x_ref[...].reshape(B_LOC * SQ, D_MODEL).astype(jnp.bfloat16)

    qi = lax.broadcasted_iota(jnp.int32, (SQ, SKV), 0)
    ki = lax.broadcasted_iota(jnp.int32, (SQ, SKV), 1)
    mask = (jnp.abs(qi - ki) <= 128) | (ki < 32) | (qi < 32)
    bias = jnp.where(mask, 0.0, -1e9).astype(jnp.float32)

    def attn_part(p, wq):
        q = lax.dot_general(xb, wq, (((1,), (0,)), ((), ())),
                            preferred_element_type=jnp.float32)
        qb = (q * 0.125).astype(jnp.bfloat16)
        ctxbs = []
        for b in range(B_LOC):
            kb = kperm[b, :, p * D_CHUNK:(p + 1) * D_CHUNK]
            vb = vperm[b, :, p * D_CHUNK:(p + 1) * D_CHUNK]
            ctx_parts = []
            for hh in range(HQ):
                qs = qb[b * SQ:(b + 1) * SQ, hh * DH:(hh + 1) * DH]
                k = kb[:, hh * DH:(hh + 1) * DH]
                s = lax.dot_general(qs, k, (((1,), (1,)), ((), ())),
                                    preferred_element_type=jnp.float32)
                w = jnp.exp(s + bias)
                den = jnp.sum(w, axis=1, keepdims=True)
                wb = w.astype(jnp.bfloat16)
                v = vb[:, hh * DH:(hh + 1) * DH]
                ctx = lax.dot_general(wb, v, (((1,), (0,)), ((), ())),
                                      preferred_element_type=jnp.float32)
                ctx_parts.append((ctx / den).astype(jnp.bfloat16))
            ctxbs.append(jnp.concatenate(ctx_parts, axis=1))
        return ctxbs

    def out_part(p, wo, ctxbs):
        for b in range(B_LOC):
            o = lax.dot_general(ctxbs[b], wo, (((1,), (0,)), ((), ())),
                                preferred_element_type=jnp.float32)
            if p == 0:
                out_ref[b] = o
            else:
                out_ref[b] += o

    out_part(0, wob[...], attn_part(0, wqb[...]))
    send_chunk(2, diag)

    def recv_wait(comm, ssem, rsem, slot):
        recv = pltpu.make_async_remote_copy(
            src_ref=wqb if comm is commq else wob,
            dst_ref=comm.at[slot],
            send_sem=ssem.at[slot],
            recv_sem=rsem.at[slot],
            device_id=(left,),
            device_id_type=pl.DeviceIdType.MESH,
        )
        recv.wait_recv()

    for p in range(1, N_DEV):
        slot = p - 1
        recv_wait(commq, sendq, recvq, slot)
        ctxbs = attn_part(p, commq[slot])
        recv_wait(commo, sendo, recvo, slot)
        out_part(p, commo[slot], ctxbs)

    for rdma in sends:
        rdma.wait_send()


def kernel(x, Wq, K_ext, V_ext, Wo):
    my = lax.axis_index("i")

    def prep(t):
        t = lax.dynamic_slice_in_dim(t, my * B_LOC, B_LOC, 0)
        return t.astype(jnp.bfloat16).reshape(B_LOC, SKV, N_DEV * D_CHUNK)

    k2 = prep(K_ext)
    v2 = prep(V_ext)

    return pl.pallas_call(
        _body,
        out_shape=jax.ShapeDtypeStruct((B_LOC, SQ, D_MODEL), jnp.float32),
        in_specs=[
            pl.BlockSpec(memory_space=pltpu.VMEM),
            pl.BlockSpec(memory_space=pltpu.VMEM),
            pl.BlockSpec(memory_space=pltpu.VMEM),
            pl.BlockSpec(memory_space=pltpu.VMEM),
            pl.BlockSpec(memory_space=pltpu.VMEM),
        ],
        out_specs=pl.BlockSpec(memory_space=pltpu.VMEM),
        scratch_shapes=[
            pltpu.VMEM((D_MODEL, D_CHUNK), jnp.bfloat16),
            pltpu.VMEM((D_CHUNK, D_MODEL), jnp.bfloat16),
            pltpu.VMEM((3, D_MODEL, D_CHUNK), jnp.bfloat16),
            pltpu.VMEM((3, D_CHUNK, D_MODEL), jnp.bfloat16),
            pltpu.VMEM((B_LOC, SKV, N_DEV * D_CHUNK), jnp.bfloat16),
            pltpu.VMEM((B_LOC, SKV, N_DEV * D_CHUNK), jnp.bfloat16),
            pltpu.SemaphoreType.DMA((3,)),
            pltpu.SemaphoreType.DMA((3,)),
            pltpu.SemaphoreType.DMA((3,)),
            pltpu.SemaphoreType.DMA((3,)),
        ],
        compiler_params=pltpu.CompilerParams(collective_id=0),
    )(x, Wq, Wo, k2, v2)
